# baseline (speedup 1.0000x reference)
import dataclasses
import functools

import jax
import jax.numpy as jnp
from jax import lax
from jax.experimental import pallas as pl
from jax.experimental.pallas import tpu as pltpu
from jax.experimental.pallas import tpu_sc as plsc

N_NODES_C = 10000
N_PAD = 10240
E_BLK = 128
N_TEC = 32
ROW_BLK = 1024


def _sc_params(layout_passes=True, tc_tiling=None):
    cp = pltpu.CompilerParams()
    fields = pltpu.CompilerParams.__dataclass_fields__
    if not layout_passes and "needs_layout_passes" in fields:
        cp = dataclasses.replace(cp, needs_layout_passes=False)
    if tc_tiling is not None and "use_tc_tiling_on_sc" in fields:
        cp = dataclasses.replace(cp, use_tc_tiling_on_sc=tc_tiling)
    return cp


_MESH = plsc.VectorSubcoreMesh(core_axis_name="c", subcore_axis_name="s")


def _mm_body(n_chunks, acw, al_cols, x_ref, w_ref, as_ref, ad_ref,
             *out_refs):
    h = jnp.dot(x_ref[...], w_ref[...], preferred_element_type=jnp.float32)
    for c in range(n_chunks):
        out_refs[c][...] = h[:, c * acw:(c + 1) * acw]
    out_refs[n_chunks][...] = jnp.dot(h, as_ref[...],
                                      preferred_element_type=jnp.float32)
    out_refs[n_chunks + 1][...] = jnp.dot(h, ad_ref[...],
                                          preferred_element_type=jnp.float32)


def _project(x, W, A_s, A_d, acw):
    n, k = x.shape
    m = W.shape[1]
    n_chunks = m // acw
    al_cols = A_s.shape[1]
    grid = (n // ROW_BLK,)
    out_shape = ([jax.ShapeDtypeStruct((n, acw), jnp.float32)] * n_chunks
                 + [jax.ShapeDtypeStruct((n, al_cols), jnp.float32)] * 2)
    out_specs = ([pl.BlockSpec((ROW_BLK, acw), lambda i: (i, 0))] * n_chunks
                 + [pl.BlockSpec((ROW_BLK, al_cols), lambda i: (i, 0))] * 2)
    outs = pl.pallas_call(
        functools.partial(_mm_body, n_chunks, acw, al_cols),
        grid=grid,
        in_specs=[
            pl.BlockSpec((ROW_BLK, k), lambda i: (i, 0)),
            pl.BlockSpec((k, m), lambda i: (0, 0)),
            pl.BlockSpec((m, al_cols), lambda i: (0, 0)),
            pl.BlockSpec((m, al_cols), lambda i: (0, 0)),
        ],
        out_specs=out_specs,
        out_shape=out_shape,
    )(x, W, A_s, A_d)
    return outs[:n_chunks], outs[n_chunks], outs[n_chunks + 1]


def _sc_alpha_h8(als, ald, src2d, dst2d, e_pad):
    nblk = e_pad // (N_TEC * E_BLK)
    rpt = N_PAD // 16

    out_type = [
        jax.ShapeDtypeStruct((e_pad // E_BLK, E_BLK, 16), jnp.float32),
        jax.ShapeDtypeStruct((2, N_PAD, 16), jnp.float32),
    ]
    scratch = [
        pltpu.VMEM((nblk, E_BLK), jnp.int32),
        pltpu.VMEM((nblk, E_BLK), jnp.int32),
        pltpu.VMEM((E_BLK, 16), jnp.float32),
        pltpu.VMEM((E_BLK, 16), jnp.float32),
        pltpu.VMEM((E_BLK, 16), jnp.float32),
        pltpu.VMEM((E_BLK, 16), jnp.float32),
        pltpu.VMEM((E_BLK, 16), jnp.float32),
        pltpu.VMEM((E_BLK, 16), jnp.float32),
        pltpu.VMEM_SHARED((N_PAD, 16), jnp.float32),
        pltpu.SemaphoreType.DMA,
        pltpu.SemaphoreType.DMA,
    ]

    @functools.partial(pl.kernel, out_type=out_type, mesh=_MESH,
                       scratch_types=scratch,
                       compiler_params=_sc_params(layout_passes=False,
                                                  tc_tiling=False))
    def k(als_h, ald_h, src_h, dst_h, w_h, den_h,
          src_a, dst_a, g1a, g1b, g2a, g2b, wva, wvb, den_sp, gs0, gs1):
        g1 = (g1a, g1b)
        g2 = (g2a, g2b)
        w_v = (wva, wvb)
        gsem = (gs0, gs1)
        cid = lax.axis_index("c")
        sid = lax.axis_index("s")
        wid = cid * 16 + sid
        blk0 = wid * nblk

        zero16 = jnp.zeros((16,), jnp.float32)

        @pl.loop(0, E_BLK)
        def _(i):
            wva[i] = zero16

        @pl.loop(0, rpt // E_BLK)
        def _(j):
            pltpu.sync_copy(wva, den_sp.at[pl.ds(sid * rpt + j * E_BLK,
                                                 E_BLK)])

        pltpu.sync_copy(src_h.at[pl.ds(blk0, nblk)], src_a)
        pltpu.sync_copy(dst_h.at[pl.ds(blk0, nblk)], dst_a)
        plsc.subcore_barrier()

        def issue(jj, s):
            return [pltpu.async_copy(als_h.at[src_a.at[jj]], g1[s], gsem[s]),
                    pltpu.async_copy(ald_h.at[dst_a.at[jj]], g2[s], gsem[s])]

        def run(jj, s):
            @pl.loop(0, E_BLK, unroll=4)
            def _(i):
                v = g1[s][i] + g2[s][i]
                v = jnp.maximum(v, 0.0) + 0.2 * jnp.minimum(v, 0.0)
                w_v[s][i] = jnp.exp(v)

            pltpu.sync_copy(w_v[s], w_h.at[blk0 + jj])
            pltpu.sync_copy(w_v[s], den_sp.at[dst_a.at[jj]], add=True)

        @pl.loop(0, nblk, step=2)
        def _(j):
            da = issue(j, 0)
            db = issue(j + 1, 1)
            for d in da:
                d.wait()
            run(j, 0)
            for d in db:
                d.wait()
            run(j + 1, 1)

        plsc.subcore_barrier()

        @pl.loop(0, rpt // E_BLK)
        def _(j):
            r0 = sid * rpt + j * E_BLK
            pltpu.sync_copy(den_sp.at[pl.ds(r0, E_BLK)],
                            den_h.at[cid].at[pl.ds(r0, E_BLK)])

    return k(als, ald, src2d, dst2d)


def _sc_alpha_h1(als1, ald1, src2d, dst2d, e_pad):
    nblk = e_pad // (N_TEC * E_BLK)
    rpt = N_PAD // 16

    out_type = [
        jax.ShapeDtypeStruct((e_pad // E_BLK, E_BLK), jnp.float32),
        jax.ShapeDtypeStruct((2, N_PAD), jnp.float32),
    ]
    scratch = [
        pltpu.VMEM((nblk, E_BLK), jnp.int32),
        pltpu.VMEM((nblk, E_BLK), jnp.int32),
        pltpu.VMEM((N_PAD,), jnp.float32),
        pltpu.VMEM((N_PAD,), jnp.float32),
        pltpu.VMEM((E_BLK,), jnp.float32),
        pltpu.VMEM_SHARED((N_PAD,), jnp.float32),
    ]

    @functools.partial(pl.kernel, out_type=out_type, mesh=_MESH,
                       scratch_types=scratch,
                       compiler_params=_sc_params(layout_passes=False,
                                                  tc_tiling=False))
    def k(als_h, ald_h, src_h, dst_h, w_h, den_h,
          src_a, dst_a, als_t, ald_t, w_v, den_sp):
        cid = lax.axis_index("c")
        sid = lax.axis_index("s")
        wid = cid * 16 + sid
        blk0 = wid * nblk

        zero16 = jnp.zeros((16,), jnp.float32)

        @pl.loop(0, E_BLK // 16)
        def _(i):
            w_v[pl.ds(i * 16, 16)] = zero16

        @pl.loop(0, rpt // E_BLK)
        def _(j):
            pltpu.sync_copy(w_v, den_sp.at[pl.ds(sid * rpt + j * E_BLK,
                                                 E_BLK)])

        pltpu.sync_copy(src_h.at[pl.ds(blk0, nblk)], src_a)
        pltpu.sync_copy(dst_h.at[pl.ds(blk0, nblk)], dst_a)
        pltpu.sync_copy(als_h, als_t)
        pltpu.sync_copy(ald_h, ald_t)
        plsc.subcore_barrier()

        @pl.loop(0, nblk)
        def _(j):
            for v in range(E_BLK // 16):
                sv = src_a[j, pl.ds(v * 16, 16)]
                dv = dst_a[j, pl.ds(v * 16, 16)]
                a = plsc.load_gather(als_t, [sv])
                b = plsc.load_gather(ald_t, [dv])
                s = a + b
                s = jnp.maximum(s, 0.0) + 0.2 * jnp.minimum(s, 0.0)
                w_v[pl.ds(v * 16, 16)] = jnp.exp(s)
            pltpu.sync_copy(w_v, w_h.at[blk0 + j])
            pltpu.sync_copy(w_v, den_sp.at[dst_a.at[j]], add=True)

        plsc.subcore_barrier()

        @pl.loop(0, rpt // E_BLK)
        def _(j):
            r0 = sid * rpt + j * E_BLK
            pltpu.sync_copy(den_sp.at[pl.ds(r0, E_BLK)],
                            den_h.at[cid].at[pl.ds(r0, E_BLK)])

    return k(als1, ald1, src2d, dst2d)


def _sc_agg(h_chunks, w, src2d, dst2d, e_pad, hpc, ceff, vcols, wdim):
    acw = h_chunks[0].shape[1]
    n_chunks = len(h_chunks)
    nblk = e_pad // (N_TEC * E_BLK)
    rpt = N_PAD // 16

    out_type = [jax.ShapeDtypeStruct((2, N_PAD, acw), jnp.float32)
                for _ in range(n_chunks)]
    wshape = (E_BLK, 16) if wdim == 16 else (E_BLK,)
    scratch = [
        pltpu.VMEM((nblk, E_BLK), jnp.int32),
        pltpu.VMEM((nblk, E_BLK), jnp.int32),
        pltpu.VMEM(wshape, jnp.float32),
        pltpu.VMEM(wshape, jnp.float32),
        pltpu.VMEM((E_BLK, acw), jnp.float32),
        pltpu.VMEM((E_BLK, acw), jnp.float32),
        pltpu.VMEM((E_BLK, acw), jnp.float32),
        pltpu.VMEM_SHARED((N_PAD, acw), jnp.float32),
        pltpu.SemaphoreType.DMA,
        pltpu.SemaphoreType.DMA,
    ]

    @functools.partial(pl.kernel, out_type=out_type, mesh=_MESH,
                       scratch_types=scratch,
                       compiler_params=_sc_params(layout_passes=False,
                                                  tc_tiling=False))
    def k(*refs):
        h_refs = refs[:n_chunks]
        w_h = refs[n_chunks]
        src_h = refs[n_chunks + 1]
        dst_h = refs[n_chunks + 2]
        out_hs = refs[n_chunks + 3:2 * n_chunks + 3]
        (src_a, dst_a, wva, wvb, gba, gbb, zbuf, out_sp, gs0, gs1) = (
            refs[2 * n_chunks + 3:])
        w_v = (wva, wvb)
        gbuf = (gba, gbb)
        gsem = (gs0, gs1)

        cid = lax.axis_index("c")
        sid = lax.axis_index("s")
        wid = cid * 16 + sid
        blk0 = wid * nblk

        zero16 = jnp.zeros((16,), jnp.float32)

        @pl.loop(0, E_BLK)
        def _(i):
            for v in range(acw // 16):
                zbuf[i, pl.ds(v * 16, 16)] = zero16

        @pl.loop(0, rpt // E_BLK)
        def _(j):
            pltpu.sync_copy(zbuf, out_sp.at[pl.ds(sid * rpt + j * E_BLK,
                                                  E_BLK)])

        pltpu.sync_copy(src_h.at[pl.ds(blk0, nblk)], src_a)
        pltpu.sync_copy(dst_h.at[pl.ds(blk0, nblk)], dst_a)
        plsc.subcore_barrier()

        for c in range(n_chunks):
            def issue(jj, s, _c=c):
                return [
                    pltpu.async_copy(w_h.at[blk0 + jj], w_v[s], gsem[s]),
                    pltpu.async_copy(h_refs[_c].at[src_a.at[jj]], gbuf[s],
                                     gsem[s]),
                ]

            def run(jj, s, _c=c):
                @pl.loop(0, E_BLK, unroll=4)
                def _(i):
                    for kk in range(vcols // ceff):
                        hidx = _c * hpc + kk
                        if wdim == 16:
                            spl = plsc.load_gather(
                                w_v[s], [jnp.full((16,), i, jnp.int32),
                                         jnp.full((16,), hidx, jnp.int32)])
                        else:
                            spl = plsc.load_gather(
                                w_v[s], [jnp.full((16,), i, jnp.int32)])
                        for v in range(ceff // 16):
                            off = kk * ceff + v * 16
                            gbuf[s][i, pl.ds(off, 16)] = (
                                gbuf[s][i, pl.ds(off, 16)] * spl)

                return pltpu.async_copy(gbuf[s], out_sp.at[dst_a.at[jj]],
                                        gsem[s], add=True)

            @pl.loop(0, nblk, step=2)
            def _(j):
                da = issue(j, 0)
                db = issue(j + 1, 1)
                for d in da:
                    d.wait()
                sa = run(j, 0)
                for d in db:
                    d.wait()
                sb = run(j + 1, 1)
                sa.wait()
                sb.wait()

            plsc.subcore_barrier()

            @pl.loop(0, rpt // E_BLK)
            def _(j):
                r0 = sid * rpt + j * E_BLK
                pltpu.sync_copy(out_sp.at[pl.ds(r0, E_BLK)],
                                out_hs[c].at[cid].at[pl.ds(r0, E_BLK)])
                if c < n_chunks - 1:
                    pltpu.sync_copy(zbuf, out_sp.at[pl.ds(r0, E_BLK)])

            if c < n_chunks - 1:
                plsc.subcore_barrier()

    return list(k(*h_chunks, w, src2d, dst2d))


def _epi_body(n_chunks, acw, d_out, hpc, ceff, dd, relu, den_ref, b_ref,
              *refs):
    part_refs = refs[:n_chunks]
    o_ref = refs[n_chunks]
    den = den_ref[0] + den_ref[1] + 1e-16
    for c in range(n_chunks):
        p = part_refs[c][0] + part_refs[c][1]
        vcols = min(d_out - c * acw, acw)
        for kk in range(vcols // ceff):
            hidx = (c * hpc + kk) if dd == 16 else 0
            col0 = c * acw + kk * ceff
            seg = (p[:, kk * ceff:(kk + 1) * ceff]
                   / den[:, hidx:hidx + 1]
                   + b_ref[0, col0:col0 + ceff])
            if relu:
                seg = jnp.maximum(seg, 0.0)
            o_ref[:, col0:col0 + ceff] = seg


def _epilogue(den, parts, b_row, d_out, hpc, ceff, dd, relu):
    n_chunks = len(parts)
    acw = parts[0].shape[2]
    grid = (N_PAD // ROW_BLK,)
    return pl.pallas_call(
        functools.partial(_epi_body, n_chunks, acw, d_out, hpc, ceff, dd,
                          relu),
        grid=grid,
        in_specs=[
            pl.BlockSpec((2, ROW_BLK, dd), lambda i: (0, i, 0)),
            pl.BlockSpec((1, d_out), lambda i: (0, 0)),
        ] + [pl.BlockSpec((2, ROW_BLK, acw), lambda i: (0, i, 0))] * n_chunks,
        out_specs=pl.BlockSpec((ROW_BLK, d_out), lambda i: (i, 0)),
        out_shape=jax.ShapeDtypeStruct((N_PAD, d_out), jnp.float32),
    )(den, b_row, *parts)


def _attn_mat(a, d_model, out_cols):
    heads, ch = a.shape
    A = jnp.zeros((d_model, out_cols), jnp.float32)
    for h in range(heads):
        A = A.at[h * ch:h * ch + ch, h].set(a[h])
    return A


def _gat_layer(x_pad, src2d, dst2d, e_pad, W, a_s, a_d, b, d_out, acw,
               relu):
    heads, ch = a_s.shape
    d_pad = W.shape[1]
    if heads > 1:
        al_cols, dd, wdim = 16, 16, 16
        hpc = acw // ch
        ceff = ch
    else:
        al_cols, dd, wdim = 1, 1, 1
        hpc = 1
        ceff = min(d_out, acw)
    A_s = _attn_mat(a_s, d_pad, al_cols)
    A_d = _attn_mat(a_d, d_pad, al_cols)
    h_chunks, als, ald = _project(x_pad, W, A_s, A_d, acw)
    if heads > 1:
        w, den = _sc_alpha_h8(als, ald, src2d, dst2d, e_pad)
    else:
        w, den = _sc_alpha_h1(als.reshape(N_PAD), ald.reshape(N_PAD),
                              src2d, dst2d, e_pad)
        den = den.reshape(2, N_PAD, 1)
    parts = _sc_agg(h_chunks, w, src2d, dst2d, e_pad, hpc, ceff,
                    min(d_out, acw), wdim)
    b_row = b.reshape(1, d_out)
    return _epilogue(den, parts, b_row, d_out, hpc, ceff, dd, relu)


def kernel(x, edge_idx, W1, a_src1, a_dst1, b1, W2, a_src2, a_dst2, b2,
           W3, a_src3, a_dst3, b3):
    n = x.shape[0]
    loop = jnp.arange(n, dtype=jnp.int32)
    src = jnp.concatenate([edge_idx[0].astype(jnp.int32), loop])
    dst = jnp.concatenate([edge_idx[1].astype(jnp.int32), loop])
    e = src.shape[0]
    quantum = N_TEC * E_BLK * 8
    e_pad = ((e + quantum - 1) // quantum) * quantum
    pad_idx = n + (jnp.arange(e_pad - e, dtype=jnp.int32) % (N_PAD - n))
    src = jnp.concatenate([src, pad_idx]).reshape(-1, E_BLK)
    dst = jnp.concatenate([dst, pad_idx]).reshape(-1, E_BLK)

    x_pad = jnp.pad(x, ((0, N_PAD - n), (0, 0)))

    W3p = jnp.pad(W3, ((0, 0), (0, 16 - W3.shape[1])))
    b3p = jnp.pad(b3, (0, 16 - b3.shape[0]))

    h = _gat_layer(x_pad, src, dst, e_pad, W1, a_src1, a_dst1, b1, 512, 64,
                   True)
    h = _gat_layer(h, src, dst, e_pad, W2, a_src2, a_dst2, b2, 64, 64, True)
    out = _gat_layer(h, src, dst, e_pad, W3p, a_src3, a_dst3, b3p, 16, 16,
                     False)
    return out[:n, :3]


# speedup vs baseline: 1.1188x; 1.1188x over previous
import dataclasses
import functools

import jax
import jax.numpy as jnp
from jax import lax
from jax.experimental import pallas as pl
from jax.experimental.pallas import tpu as pltpu
from jax.experimental.pallas import tpu_sc as plsc

N_NODES_C = 10000
N_PAD = 10240
E_BLK = 128
N_TEC = 32
ROW_BLK = 1024


def _sc_params(layout_passes=True, tc_tiling=None):
    cp = pltpu.CompilerParams()
    fields = pltpu.CompilerParams.__dataclass_fields__
    if not layout_passes and "needs_layout_passes" in fields:
        cp = dataclasses.replace(cp, needs_layout_passes=False)
    if tc_tiling is not None and "use_tc_tiling_on_sc" in fields:
        cp = dataclasses.replace(cp, use_tc_tiling_on_sc=tc_tiling)
    return cp


_MESH = plsc.VectorSubcoreMesh(core_axis_name="c", subcore_axis_name="s")


def _mm_body(n_chunks, acw, al_cols, x_ref, w_ref, as_ref, ad_ref,
             *out_refs):
    h = jnp.dot(x_ref[...], w_ref[...], preferred_element_type=jnp.float32)
    for c in range(n_chunks):
        out_refs[c][...] = h[:, c * acw:(c + 1) * acw]
    out_refs[n_chunks][...] = jnp.dot(h, as_ref[...],
                                      preferred_element_type=jnp.float32)
    out_refs[n_chunks + 1][...] = jnp.dot(h, ad_ref[...],
                                          preferred_element_type=jnp.float32)


def _project(x, W, A_s, A_d, acw):
    n, k = x.shape
    m = W.shape[1]
    n_chunks = m // acw
    al_cols = A_s.shape[1]
    grid = (n // ROW_BLK,)
    out_shape = ([jax.ShapeDtypeStruct((n, acw), jnp.float32)] * n_chunks
                 + [jax.ShapeDtypeStruct((n, al_cols), jnp.float32)] * 2)
    out_specs = ([pl.BlockSpec((ROW_BLK, acw), lambda i: (i, 0))] * n_chunks
                 + [pl.BlockSpec((ROW_BLK, al_cols), lambda i: (i, 0))] * 2)
    outs = pl.pallas_call(
        functools.partial(_mm_body, n_chunks, acw, al_cols),
        grid=grid,
        in_specs=[
            pl.BlockSpec((ROW_BLK, k), lambda i: (i, 0)),
            pl.BlockSpec((k, m), lambda i: (0, 0)),
            pl.BlockSpec((m, al_cols), lambda i: (0, 0)),
            pl.BlockSpec((m, al_cols), lambda i: (0, 0)),
        ],
        out_specs=out_specs,
        out_shape=out_shape,
    )(x, W, A_s, A_d)
    return outs[:n_chunks], outs[n_chunks], outs[n_chunks + 1]


def _sc_alpha_h8(als, ald, src2d, dst2d, e_pad):
    nblk = e_pad // (N_TEC * E_BLK)
    rpt = N_PAD // 16

    out_type = [
        jax.ShapeDtypeStruct((e_pad // E_BLK, E_BLK, 16), jnp.float32),
        jax.ShapeDtypeStruct((2, N_PAD, 16), jnp.float32),
    ]
    scratch = [
        pltpu.VMEM((nblk, E_BLK), jnp.int32),
        pltpu.VMEM((nblk, E_BLK), jnp.int32),
        pltpu.VMEM((E_BLK, 16), jnp.float32),
        pltpu.VMEM((E_BLK, 16), jnp.float32),
        pltpu.VMEM((E_BLK, 16), jnp.float32),
        pltpu.VMEM((E_BLK, 16), jnp.float32),
        pltpu.VMEM((E_BLK, 16), jnp.float32),
        pltpu.VMEM((E_BLK, 16), jnp.float32),
        pltpu.VMEM_SHARED((N_PAD, 16), jnp.float32),
        pltpu.SemaphoreType.DMA,
        pltpu.SemaphoreType.DMA,
    ]

    @functools.partial(pl.kernel, out_type=out_type, mesh=_MESH,
                       scratch_types=scratch,
                       compiler_params=_sc_params(layout_passes=False,
                                                  tc_tiling=False))
    def k(als_h, ald_h, src_h, dst_h, w_h, den_h,
          src_a, dst_a, g1a, g1b, g2a, g2b, wva, wvb, den_sp, gs0, gs1):
        g1 = (g1a, g1b)
        g2 = (g2a, g2b)
        w_v = (wva, wvb)
        gsem = (gs0, gs1)
        cid = lax.axis_index("c")
        sid = lax.axis_index("s")
        wid = cid * 16 + sid
        blk0 = wid * nblk

        zero16 = jnp.zeros((16,), jnp.float32)

        @pl.loop(0, E_BLK)
        def _(i):
            wva[i] = zero16

        @pl.loop(0, rpt // E_BLK)
        def _(j):
            pltpu.sync_copy(wva, den_sp.at[pl.ds(sid * rpt + j * E_BLK,
                                                 E_BLK)])

        pltpu.sync_copy(src_h.at[pl.ds(blk0, nblk)], src_a)
        pltpu.sync_copy(dst_h.at[pl.ds(blk0, nblk)], dst_a)
        plsc.subcore_barrier()

        def issue(jj, s):
            return [pltpu.async_copy(als_h.at[src_a.at[jj]], g1[s], gsem[s]),
                    pltpu.async_copy(ald_h.at[dst_a.at[jj]], g2[s], gsem[s])]

        def run(jj, s):
            @pl.loop(0, E_BLK)
            def _(i):
                v = g1[s][i] + g2[s][i]
                v = jnp.maximum(v, 0.0) + 0.2 * jnp.minimum(v, 0.0)
                w_v[s][i] = jnp.exp(v)

            pltpu.sync_copy(w_v[s], w_h.at[blk0 + jj])
            pltpu.sync_copy(w_v[s], den_sp.at[dst_a.at[jj]], add=True)

        @pl.loop(0, nblk, step=2)
        def _(j):
            da = issue(j, 0)
            db = issue(j + 1, 1)
            for d in da:
                d.wait()
            run(j, 0)
            for d in db:
                d.wait()
            run(j + 1, 1)

        plsc.subcore_barrier()

        @pl.loop(0, rpt // E_BLK)
        def _(j):
            r0 = sid * rpt + j * E_BLK
            pltpu.sync_copy(den_sp.at[pl.ds(r0, E_BLK)],
                            den_h.at[cid].at[pl.ds(r0, E_BLK)])

    return k(als, ald, src2d, dst2d)


def _sc_alpha_h1(als1, ald1, src2d, dst2d, e_pad):
    nblk = e_pad // (N_TEC * E_BLK)
    rpt = N_PAD // 16

    out_type = [
        jax.ShapeDtypeStruct((e_pad // E_BLK, E_BLK), jnp.float32),
        jax.ShapeDtypeStruct((2, N_PAD), jnp.float32),
    ]
    scratch = [
        pltpu.VMEM((nblk, E_BLK), jnp.int32),
        pltpu.VMEM((nblk, E_BLK), jnp.int32),
        pltpu.VMEM((N_PAD,), jnp.float32),
        pltpu.VMEM((N_PAD,), jnp.float32),
        pltpu.VMEM((E_BLK,), jnp.float32),
        pltpu.VMEM_SHARED((N_PAD,), jnp.float32),
    ]

    @functools.partial(pl.kernel, out_type=out_type, mesh=_MESH,
                       scratch_types=scratch,
                       compiler_params=_sc_params(layout_passes=False,
                                                  tc_tiling=False))
    def k(als_h, ald_h, src_h, dst_h, w_h, den_h,
          src_a, dst_a, als_t, ald_t, w_v, den_sp):
        cid = lax.axis_index("c")
        sid = lax.axis_index("s")
        wid = cid * 16 + sid
        blk0 = wid * nblk

        zero16 = jnp.zeros((16,), jnp.float32)

        @pl.loop(0, E_BLK // 16)
        def _(i):
            w_v[pl.ds(i * 16, 16)] = zero16

        @pl.loop(0, rpt // E_BLK)
        def _(j):
            pltpu.sync_copy(w_v, den_sp.at[pl.ds(sid * rpt + j * E_BLK,
                                                 E_BLK)])

        pltpu.sync_copy(src_h.at[pl.ds(blk0, nblk)], src_a)
        pltpu.sync_copy(dst_h.at[pl.ds(blk0, nblk)], dst_a)
        pltpu.sync_copy(als_h, als_t)
        pltpu.sync_copy(ald_h, ald_t)
        plsc.subcore_barrier()

        @pl.loop(0, nblk)
        def _(j):
            for v in range(E_BLK // 16):
                sv = src_a[j, pl.ds(v * 16, 16)]
                dv = dst_a[j, pl.ds(v * 16, 16)]
                a = plsc.load_gather(als_t, [sv])
                b = plsc.load_gather(ald_t, [dv])
                s = a + b
                s = jnp.maximum(s, 0.0) + 0.2 * jnp.minimum(s, 0.0)
                w_v[pl.ds(v * 16, 16)] = jnp.exp(s)
            pltpu.sync_copy(w_v, w_h.at[blk0 + j])
            pltpu.sync_copy(w_v, den_sp.at[dst_a.at[j]], add=True)

        plsc.subcore_barrier()

        @pl.loop(0, rpt // E_BLK)
        def _(j):
            r0 = sid * rpt + j * E_BLK
            pltpu.sync_copy(den_sp.at[pl.ds(r0, E_BLK)],
                            den_h.at[cid].at[pl.ds(r0, E_BLK)])

    return k(als1, ald1, src2d, dst2d)


def _sc_agg(h_chunks, w, src2d, dst2d, e_pad, hpc, ceff, vcols, wdim):
    acw = h_chunks[0].shape[1]
    n_chunks = len(h_chunks)
    nblk = e_pad // (N_TEC * E_BLK)
    rpt = N_PAD // 16

    out_type = [jax.ShapeDtypeStruct((2, N_PAD, acw), jnp.float32)
                for _ in range(n_chunks)]
    wshape = (E_BLK, 16) if wdim == 16 else (E_BLK,)
    scratch = [
        pltpu.VMEM((nblk, E_BLK), jnp.int32),
        pltpu.VMEM((nblk, E_BLK), jnp.int32),
        pltpu.VMEM(wshape, jnp.float32),
        pltpu.VMEM(wshape, jnp.float32),
        pltpu.VMEM(wshape, jnp.float32),
        pltpu.VMEM(wshape, jnp.float32),
        pltpu.VMEM((E_BLK, acw), jnp.float32),
        pltpu.VMEM((E_BLK, acw), jnp.float32),
        pltpu.VMEM((E_BLK, acw), jnp.float32),
        pltpu.VMEM((E_BLK, acw), jnp.float32),
        pltpu.VMEM((E_BLK, acw), jnp.float32),
        pltpu.VMEM_SHARED((N_PAD, acw), jnp.float32),
        pltpu.SemaphoreType.DMA,
        pltpu.SemaphoreType.DMA,
        pltpu.SemaphoreType.DMA,
        pltpu.SemaphoreType.DMA,
    ]

    @functools.partial(pl.kernel, out_type=out_type, mesh=_MESH,
                       scratch_types=scratch,
                       compiler_params=_sc_params(layout_passes=False,
                                                  tc_tiling=False))
    def k(*refs):
        h_refs = refs[:n_chunks]
        w_h = refs[n_chunks]
        src_h = refs[n_chunks + 1]
        dst_h = refs[n_chunks + 2]
        out_hs = refs[n_chunks + 3:2 * n_chunks + 3]
        (src_a, dst_a, wva, wvb, wvc, wvd, gba, gbb, gbc, gbd, zbuf,
         out_sp, gs0, gs1, gs2, gs3) = refs[2 * n_chunks + 3:]
        w_v = (wva, wvb, wvc, wvd)
        gbuf = (gba, gbb, gbc, gbd)
        gsem = (gs0, gs1, gs2, gs3)

        cid = lax.axis_index("c")
        sid = lax.axis_index("s")
        wid = cid * 16 + sid
        blk0 = wid * nblk

        zero16 = jnp.zeros((16,), jnp.float32)

        @pl.loop(0, E_BLK)
        def _(i):
            for v in range(acw // 16):
                zbuf[i, pl.ds(v * 16, 16)] = zero16

        @pl.loop(0, rpt // E_BLK)
        def _(j):
            pltpu.sync_copy(zbuf, out_sp.at[pl.ds(sid * rpt + j * E_BLK,
                                                  E_BLK)])

        pltpu.sync_copy(src_h.at[pl.ds(blk0, nblk)], src_a)
        pltpu.sync_copy(dst_h.at[pl.ds(blk0, nblk)], dst_a)
        plsc.subcore_barrier()

        for c in range(n_chunks):
            def issue(jj, s, _c=c):
                return [
                    pltpu.async_copy(w_h.at[blk0 + jj], w_v[s], gsem[s]),
                    pltpu.async_copy(h_refs[_c].at[src_a.at[jj]], gbuf[s],
                                     gsem[s]),
                ]

            def run(jj, s, _c=c):
                @pl.loop(0, E_BLK)
                def _(i):
                    for kk in range(vcols // ceff):
                        hidx = _c * hpc + kk
                        if wdim == 16:
                            spl = plsc.load_gather(
                                w_v[s], [jnp.full((16,), i, jnp.int32),
                                         jnp.full((16,), hidx, jnp.int32)])
                        else:
                            spl = plsc.load_gather(
                                w_v[s], [jnp.full((16,), i, jnp.int32)])
                        for v in range(ceff // 16):
                            off = kk * ceff + v * 16
                            gbuf[s][i, pl.ds(off, 16)] = (
                                gbuf[s][i, pl.ds(off, 16)] * spl)

                return pltpu.async_copy(gbuf[s], out_sp.at[dst_a.at[jj]],
                                        gsem[s], add=True)

            @pl.loop(0, nblk, step=4)
            def _(j):
                gds = [issue(j + t, t) for t in range(4)]
                sds = []
                for t in range(4):
                    for d in gds[t]:
                        d.wait()
                    sds.append(run(j + t, t))
                for d in sds:
                    d.wait()

            plsc.subcore_barrier()

            @pl.loop(0, rpt // E_BLK)
            def _(j):
                r0 = sid * rpt + j * E_BLK
                pltpu.sync_copy(out_sp.at[pl.ds(r0, E_BLK)],
                                out_hs[c].at[cid].at[pl.ds(r0, E_BLK)])
                if c < n_chunks - 1:
                    pltpu.sync_copy(zbuf, out_sp.at[pl.ds(r0, E_BLK)])

            if c < n_chunks - 1:
                plsc.subcore_barrier()

    return list(k(*h_chunks, w, src2d, dst2d))


def _epi_body(n_chunks, acw, d_out, hpc, ceff, dd, relu, den_ref, b_ref,
              *refs):
    part_refs = refs[:n_chunks]
    o_ref = refs[n_chunks]
    den = den_ref[0] + den_ref[1] + 1e-16
    for c in range(n_chunks):
        p = part_refs[c][0] + part_refs[c][1]
        vcols = min(d_out - c * acw, acw)
        for kk in range(vcols // ceff):
            hidx = (c * hpc + kk) if dd == 16 else 0
            col0 = c * acw + kk * ceff
            seg = (p[:, kk * ceff:(kk + 1) * ceff]
                   / den[:, hidx:hidx + 1]
                   + b_ref[0, col0:col0 + ceff])
            if relu:
                seg = jnp.maximum(seg, 0.0)
            o_ref[:, col0:col0 + ceff] = seg


def _epilogue(den, parts, b_row, d_out, hpc, ceff, dd, relu):
    n_chunks = len(parts)
    acw = parts[0].shape[2]
    grid = (N_PAD // ROW_BLK,)
    return pl.pallas_call(
        functools.partial(_epi_body, n_chunks, acw, d_out, hpc, ceff, dd,
                          relu),
        grid=grid,
        in_specs=[
            pl.BlockSpec((2, ROW_BLK, dd), lambda i: (0, i, 0)),
            pl.BlockSpec((1, d_out), lambda i: (0, 0)),
        ] + [pl.BlockSpec((2, ROW_BLK, acw), lambda i: (0, i, 0))] * n_chunks,
        out_specs=pl.BlockSpec((ROW_BLK, d_out), lambda i: (i, 0)),
        out_shape=jax.ShapeDtypeStruct((N_PAD, d_out), jnp.float32),
    )(den, b_row, *parts)


def _attn_mat(a, d_model, out_cols):
    heads, ch = a.shape
    A = jnp.zeros((d_model, out_cols), jnp.float32)
    for h in range(heads):
        A = A.at[h * ch:h * ch + ch, h].set(a[h])
    return A


def _gat_layer(x_pad, src2d, dst2d, e_pad, W, a_s, a_d, b, d_out, acw,
               relu):
    heads, ch = a_s.shape
    d_pad = W.shape[1]
    if heads > 1:
        al_cols, dd, wdim = 16, 16, 16
        hpc = acw // ch
        ceff = ch
    else:
        al_cols, dd, wdim = 1, 1, 1
        hpc = 1
        ceff = min(d_out, acw)
    A_s = _attn_mat(a_s, d_pad, al_cols)
    A_d = _attn_mat(a_d, d_pad, al_cols)
    h_chunks, als, ald = _project(x_pad, W, A_s, A_d, acw)
    if heads > 1:
        w, den = _sc_alpha_h8(als, ald, src2d, dst2d, e_pad)
    else:
        w, den = _sc_alpha_h1(als.reshape(N_PAD), ald.reshape(N_PAD),
                              src2d, dst2d, e_pad)
        den = den.reshape(2, N_PAD, 1)
    parts = _sc_agg(h_chunks, w, src2d, dst2d, e_pad, hpc, ceff,
                    min(d_out, acw), wdim)
    b_row = b.reshape(1, d_out)
    return _epilogue(den, parts, b_row, d_out, hpc, ceff, dd, relu)


def kernel(x, edge_idx, W1, a_src1, a_dst1, b1, W2, a_src2, a_dst2, b2,
           W3, a_src3, a_dst3, b3):
    n = x.shape[0]
    loop = jnp.arange(n, dtype=jnp.int32)
    src = jnp.concatenate([edge_idx[0].astype(jnp.int32), loop])
    dst = jnp.concatenate([edge_idx[1].astype(jnp.int32), loop])
    e = src.shape[0]
    quantum = N_TEC * E_BLK * 8
    e_pad = ((e + quantum - 1) // quantum) * quantum
    pad_idx = n + (jnp.arange(e_pad - e, dtype=jnp.int32) % (N_PAD - n))
    src = jnp.concatenate([src, pad_idx]).reshape(-1, E_BLK)
    dst = jnp.concatenate([dst, pad_idx]).reshape(-1, E_BLK)

    x_pad = jnp.pad(x, ((0, N_PAD - n), (0, 0)))

    W3p = jnp.pad(W3, ((0, 0), (0, 16 - W3.shape[1])))
    b3p = jnp.pad(b3, (0, 16 - b3.shape[0]))

    h = _gat_layer(x_pad, src, dst, e_pad, W1, a_src1, a_dst1, b1, 512, 64,
                   True)
    h = _gat_layer(h, src, dst, e_pad, W2, a_src2, a_dst2, b2, 64, 64, True)
    out = _gat_layer(h, src, dst, e_pad, W3p, a_src3, a_dst3, b3p, 16, 16,
                     False)
    return out[:n, :3]
